# baseline (speedup 1.0000x reference)
import functools

import jax
import jax.numpy as jnp
from jax import lax
from jax.experimental import pallas as pl
from jax.experimental.pallas import tpu as pltpu
from jax.experimental.pallas import tpu_sc as plsc

VOCAB = 4096
D = 4096
NC, NS = 2, 16
NW = NC * NS
B = 4 * 2048
B_PER_W = B // NW
CHUNK = 8
N_CHUNKS = B_PER_W // CHUNK
NBUF = 3

_mesh = plsc.VectorSubcoreMesh(
    core_axis_name="c", subcore_axis_name="s", num_cores=NC, num_subcores=NS
)


@functools.partial(
    pl.kernel,
    out_type=jax.ShapeDtypeStruct((B, D), jnp.float32),
    mesh=_mesh,
    scratch_types=[
        pltpu.VMEM((B_PER_W,), jnp.int32),
        pltpu.VMEM((CHUNK, D), jnp.float32),
        pltpu.VMEM((CHUNK, D), jnp.float32),
        pltpu.VMEM((CHUNK, D), jnp.float32),
        pltpu.SemaphoreType.DMA,
        pltpu.SemaphoreType.DMA,
        pltpu.SemaphoreType.DMA,
        pltpu.SemaphoreType.DMA,
        pltpu.SemaphoreType.DMA,
        pltpu.SemaphoreType.DMA,
    ],
)
def _gather_rows(idx_hbm, table_hbm, out_hbm, idx_v, rows0, rows1, rows2,
                 g0, g1, g2, w0, w1, w2):
    wid = lax.axis_index("s") * NC + lax.axis_index("c")
    base = wid * B_PER_W
    pltpu.sync_copy(idx_hbm.at[pl.ds(base, B_PER_W)], idx_v)

    bufs, gsems, wsems = (rows0, rows1, rows2), (g0, g1, g2), (w0, w1, w2)

    def gather_desc(j, b):
        return pltpu.make_async_copy(
            table_hbm.at[idx_v.at[pl.ds(j * CHUNK, CHUNK)]], bufs[b], gsems[b]
        )

    def write_desc(j, b):
        return pltpu.make_async_copy(
            bufs[b], out_hbm.at[pl.ds(base + j * CHUNK, CHUNK)], wsems[b]
        )

    gather_desc(0, 0).start()
    gather_desc(1, 1).start()

    gather_desc(0, 0).wait()
    write_desc(0, 0).start()
    gather_desc(2, 2).start()

    gather_desc(1, 1).wait()
    write_desc(1, 1).start()
    write_desc(0, 0).wait()
    gather_desc(3, 0).start()

    gather_desc(2, 2).wait()
    write_desc(2, 2).start()
    write_desc(1, 1).wait()
    gather_desc(4, 1).start()

    @pl.loop(3, N_CHUNKS - 2, step=3)
    def _trip(i):
        for u in range(3):
            j = i + u
            b = u
            gather_desc(j, b).wait()
            write_desc(j, b).start()
            write_desc(j - 1, (b + 2) % 3).wait()
            gather_desc(j + 2, (b + 2) % 3).start()

    for j in (N_CHUNKS - 2, N_CHUNKS - 1):
        b = j % 3
        gather_desc(j, b).wait()
        write_desc(j, b).start()
    for j in (N_CHUNKS - 3, N_CHUNKS - 2, N_CHUNKS - 1):
        write_desc(j, j % 3).wait()


def kernel(idx, embedding):
    flat = idx.reshape(-1).astype(jnp.int32)
    out = _gather_rows(flat, embedding)
    return out.reshape(idx.shape + (VOCAB,))
